# baseline (speedup 1.0000x reference)
import functools

import jax
import jax.numpy as jnp
from jax import lax
from jax.experimental import pallas as pl
from jax.experimental.pallas import tpu as pltpu
from jax.experimental.pallas import tpu_sc as plsc

N = 10000
NPAD = 10240
E = 160000
EPAD = 160256
T = 12
H = 32
F = 128
HQ = 96
TQ = 3
NC, NS, L = 2, 16, 16

@functools.cache
def _mesh():
  return plsc.VectorSubcoreMesh(
      core_axis_name="c", subcore_axis_name="s", num_cores=NC, num_subcores=NS)


_f32 = jnp.float32
_i32 = jnp.int32

_SC_PARAMS = pltpu.CompilerParams(
    needs_layout_passes=False, use_tc_tiling_on_sc=False)


def _deg_body(dst4, ew4, degp, dstv, ewv, degv, kbuf):
  cid = lax.axis_index("c")
  sid = lax.axis_index("s")
  wid = cid * NS + sid
  pltpu.sync_copy(dst4.at[wid], dstv)
  pltpu.sync_copy(ew4.at[wid], ewv)

  z16 = jnp.zeros((L,), _f32)

  @pl.loop(0, NPAD // L)
  def _zero(i):
    degv[pl.ds(i * L, L)] = z16

  lane = lax.iota(_i32, L)
  nxt = jnp.minimum(lane + 1, L - 1)

  @pl.loop(0, EPAD // (32 * L))
  def _chunk(i):
    keys = dstv[i]
    vals = ewv[i]
    sk, sv = plsc.sort_key_val(keys, vals)
    cs = plsc.cumsum(sv)
    kbuf[...] = sk
    knext = plsc.load_gather(kbuf, [nxt])
    is_last = (sk != knext) | (lane == L - 1)
    plsc.addupdate_scatter(degv, [sk], cs, mask=is_last)
    notfin = is_last & (lane < L - 1)
    plsc.addupdate_scatter(degv, [knext], -cs, mask=notfin)

  pltpu.sync_copy(degv, degp.at[wid])


def _deg_call(dst4, ew4):
  return pl.kernel(
      _deg_body,
      out_type=jax.ShapeDtypeStruct((32, NPAD), _f32),
      mesh=_mesh(),
      compiler_params=_SC_PARAMS,
      scratch_types=[
          pltpu.VMEM((EPAD // (32 * L), L), _i32),
          pltpu.VMEM((EPAD // (32 * L), L), _f32),
          pltpu.VMEM((NPAD,), _f32),
          pltpu.VMEM((L,), _i32),
      ],
  )(dst4, ew4)


def _w_body(src4, dst4, ew4, dinv_hbm, w4, srcv, dstv, ewv, dinvv, wv):
  cid = lax.axis_index("c")
  sid = lax.axis_index("s")
  wid = cid * NS + sid
  pltpu.sync_copy(dinv_hbm, dinvv)
  pltpu.sync_copy(src4.at[wid], srcv)
  pltpu.sync_copy(dst4.at[wid], dstv)
  pltpu.sync_copy(ew4.at[wid], ewv)

  @pl.loop(0, EPAD // (32 * L))
  def _chunk(i):
    s = srcv[i]
    d = dstv[i]
    e = ewv[i]
    ws = plsc.load_gather(dinvv, [s])
    wd = plsc.load_gather(dinvv, [d])
    wv[i] = ws * e * wd

  pltpu.sync_copy(wv, w4.at[wid])


def _w_call(src4, dst4, ew4, dinv):
  return pl.kernel(
      _w_body,
      out_type=jax.ShapeDtypeStruct((32, EPAD // (32 * L), L), _f32),
      mesh=_mesh(),
      compiler_params=_SC_PARAMS,
      scratch_types=[
          pltpu.VMEM((EPAD // (32 * L), L), _i32),
          pltpu.VMEM((EPAD // (32 * L), L), _i32),
          pltpu.VMEM((EPAD // (32 * L), L), _f32),
          pltpu.VMEM((NPAD,), _f32),
          pltpu.VMEM((EPAD // (32 * L), L), _f32),
      ],
  )(src4, dst4, ew4, dinv)


_EPT = E // NS
_CH = 80
_NCH = _EPT // _CH


def _scat_body(h2_hbm, src3, dst3, w3, out_hbm,
               srcv, dstv, wv, raw0, raw1, scl0, scl1,
               ig0, ig1, is0, is1, zbuf, agg_sh,
               gsem0, gsem1, ssem0, ssem1):
  cid = lax.axis_index("c")
  sid = lax.axis_index("s")
  raws = (raw0, raw1)
  scls = (scl0, scl1)
  igs = (ig0, ig1)
  iss = (is0, is1)
  gsems = (gsem0, gsem1)
  ssems = (ssem0, ssem1)
  hoff = cid * N

  z16 = jnp.zeros((L,), _f32)

  @pl.loop(0, 25)
  def _zero(i):
    for j in range(HQ // L):
      zbuf[i, pl.ds(j * L, L)] = z16

  @pl.loop(0, 25)
  def _zcopy(r):
    pltpu.sync_copy(zbuf, agg_sh.at[pl.ds(sid * 625 + r * 25, 25)])

  pltpu.sync_copy(src3.at[sid], srcv)
  pltpu.sync_copy(dst3.at[sid], dstv)
  pltpu.sync_copy(w3.at[sid], wv)
  plsc.subcore_barrier()

  def start_gather(ci, b):
    for p in range(_CH // L):
      igs[b][pl.ds(p * L, L)] = srcv[ci, pl.ds(p * L, L)] + hoff
    pltpu.make_async_copy(h2_hbm.at[igs[b]], raws[b], gsems[b]).start()

  def wait_gather(b):
    pltpu.make_async_copy(h2_hbm.at[igs[b]], raws[b], gsems[b]).wait()

  def scale(ci, b):
    @pl.loop(0, _CH, unroll=8)
    def _edges(e):
      wsp = plsc.load_gather(
          wv, [jnp.full((L,), ci, _i32), jnp.full((L,), e, _i32)])
      for j in range(HQ // L):
        sl = pl.ds(j * L, L)
        scls[b][e, sl] = raws[b][e, sl] * wsp

  def start_scatter(ci, b):
    for p in range(_CH // L):
      iss[b][pl.ds(p * L, L)] = dstv[ci, pl.ds(p * L, L)]
    pltpu.async_copy(scls[b], agg_sh.at[iss[b]], ssems[b], add=True)

  def wait_scatter(b):
    pltpu.make_async_copy(scls[b], agg_sh.at[iss[b]], ssems[b]).wait()

  start_gather(0, 0)
  start_gather(1, 1)
  for b in (0, 1):
    wait_gather(b)
    scale(b, b)
    start_gather(b + 2, b)
    start_scatter(b, b)

  @pl.loop(1, (_NCH - 5) // 2 + 1)
  def _main(g):
    for b in (0, 1):
      ci = 2 * g + b
      wait_gather(b)
      wait_scatter(b)
      scale(ci, b)
      start_gather(ci + 2, b)
      start_scatter(ci, b)

  wait_gather(0)
  wait_scatter(0)
  scale(_NCH - 3, 0)
  start_gather(_NCH - 1, 0)
  start_scatter(_NCH - 3, 0)

  wait_gather(1)
  wait_scatter(1)
  scale(_NCH - 2, 1)
  start_scatter(_NCH - 2, 1)

  wait_gather(0)
  wait_scatter(0)
  scale(_NCH - 1, 0)
  start_scatter(_NCH - 1, 0)

  wait_scatter(0)
  wait_scatter(1)
  plsc.subcore_barrier()

  pltpu.sync_copy(agg_sh.at[pl.ds(sid * 625, 625)],
                  out_hbm.at[pl.ds(cid * N + sid * 625, 625)])


def _scat_call(h2, src3, dst3, w3):
  return pl.kernel(
      _scat_body,
      out_type=jax.ShapeDtypeStruct((NC * N, HQ), _f32),
      mesh=_mesh(),
      compiler_params=_SC_PARAMS,
      scratch_types=[
          pltpu.VMEM((_NCH, _CH), _i32),
          pltpu.VMEM((_NCH, _CH), _i32),
          pltpu.VMEM((_NCH, _CH), _f32),
          pltpu.VMEM((_CH, HQ), _f32),
          pltpu.VMEM((_CH, HQ), _f32),
          pltpu.VMEM((_CH, HQ), _f32),
          pltpu.VMEM((_CH, HQ), _f32),
          pltpu.VMEM((_CH,), _i32),
          pltpu.VMEM((_CH,), _i32),
          pltpu.VMEM((_CH,), _i32),
          pltpu.VMEM((_CH,), _i32),
          pltpu.VMEM((25, HQ), _f32),
          pltpu.VMEM_SHARED((N, HQ), _f32),
          pltpu.SemaphoreType.DMA,
          pltpu.SemaphoreType.DMA,
          pltpu.SemaphoreType.DMA,
          pltpu.SemaphoreType.DMA,
      ],
  )(h2, src3, dst3, w3)


def _pre_body(x_ref, w_ref, b_ref, out_ref):
  w = w_ref[...]
  b = b_ref[...]
  for j in range(TQ):
    y = jnp.dot(x_ref[j], w, preferred_element_type=_f32) + b
    out_ref[:, j, :] = jnp.maximum(y, 0.0)


def _pre_call(xs, pre_W, pre_b, half):
  nb = 1000
  grid = (NC, N // nb)
  return pl.pallas_call(
      _pre_body,
      grid=grid,
      in_specs=[
          pl.BlockSpec((TQ, nb, F), lambda c, i: (2 * half + c, i, 0)),
          pl.BlockSpec((F, H), lambda c, i: (0, 0)),
          pl.BlockSpec((1, H), lambda c, i: (0, 0)),
      ],
      out_specs=pl.BlockSpec((nb, TQ, H),
                             lambda c, i: (c * (N // 1000) + i, 0, 0)),
      out_shape=jax.ShapeDtypeStruct((NC * N, TQ, H), _f32),
  )(xs, pre_W, pre_b)


def _dinv_body(degp_ref, out_ref):
  deg = jnp.sum(degp_ref[...], axis=0)
  good = deg > 0
  out_ref[...] = jnp.where(good, lax.rsqrt(jnp.maximum(deg, 1e-12)), 0.0)


def _dinv_call(degp):
  return pl.pallas_call(
      _dinv_body,
      out_shape=jax.ShapeDtypeStruct((NPAD // 128, 128), _f32),
  )(degp.reshape(32, NPAD // 128, 128))


def _conv_body(agg_ref, h_ref, wl_ref, wr_ref, b_ref, out_ref):
  y = (jnp.dot(agg_ref[...], wl_ref[...], preferred_element_type=_f32)
       + jnp.dot(h_ref[...], wr_ref[...], preferred_element_type=_f32)
       + b_ref[...])
  out_ref[...] = jnp.maximum(y, 0.0)


def _conv_call(agg128, h128, lin4, root4, b4):
  rb = 3000
  nrows = NC * N * HQ // 128
  return pl.pallas_call(
      _conv_body,
      grid=(nrows // rb,),
      in_specs=[
          pl.BlockSpec((rb, 128), lambda i: (i, 0)),
          pl.BlockSpec((rb, 128), lambda i: (i, 0)),
          pl.BlockSpec((128, 128), lambda i: (0, 0)),
          pl.BlockSpec((128, 128), lambda i: (0, 0)),
          pl.BlockSpec((1, 128), lambda i: (0, 0)),
      ],
      out_specs=pl.BlockSpec((rb, 128), lambda i: (i, 0)),
      out_shape=jax.ShapeDtypeStruct((nrows, 128), _f32),
  )(agg128, h128, lin4, root4, b4)


def _gru_body(q0_ref, q1_ref, q2_ref, q3_ref, wir_ref, wiz_ref, win_ref,
              whr_ref, whz_ref, whn_ref, bi_ref, bh_ref, out_ref):
  nb = q0_ref.shape[0]
  qs = [r[...].reshape(nb, TQ, H) for r in (q0_ref, q1_ref, q2_ref, q3_ref)]
  bi = bi_ref[...]
  bh = bh_ref[...]
  h = jnp.zeros((nb, H), _f32)
  for t in range(T):
    xt = qs[t // TQ][:, t % TQ, :]
    i_r = jnp.dot(xt, wir_ref[...], preferred_element_type=_f32) + bi[0:1]
    i_z = jnp.dot(xt, wiz_ref[...], preferred_element_type=_f32) + bi[1:2]
    i_n = jnp.dot(xt, win_ref[...], preferred_element_type=_f32) + bi[2:3]
    h_r = jnp.dot(h, whr_ref[...], preferred_element_type=_f32) + bh[0:1]
    h_z = jnp.dot(h, whz_ref[...], preferred_element_type=_f32) + bh[1:2]
    h_n = jnp.dot(h, whn_ref[...], preferred_element_type=_f32) + bh[2:3]
    r = jax.nn.sigmoid(i_r + h_r)
    z = jax.nn.sigmoid(i_z + h_z)
    c = jnp.tanh(i_n + r * h_n)
    h = (1.0 - z) * c + z * h
  out_ref[...] = h


def _gru_call(hA, hB, W_ih, W_hh, b_ih, b_hh):
  nb = 2000
  nblk = N // nb
  wir, wiz, win = W_ih[:, :H], W_ih[:, H:2 * H], W_ih[:, 2 * H:]
  whr, whz, whn = W_hh[:, :H], W_hh[:, H:2 * H], W_hh[:, 2 * H:]
  bi = b_ih.reshape(3, H)
  bh = b_hh.reshape(3, H)
  return pl.pallas_call(
      _gru_body,
      grid=(nblk,),
      in_specs=[
          pl.BlockSpec((nb, HQ), lambda i: (i, 0)),
          pl.BlockSpec((nb, HQ), lambda i: (i + nblk, 0)),
          pl.BlockSpec((nb, HQ), lambda i: (i, 0)),
          pl.BlockSpec((nb, HQ), lambda i: (i + nblk, 0)),
          pl.BlockSpec((H, H), lambda i: (0, 0)),
          pl.BlockSpec((H, H), lambda i: (0, 0)),
          pl.BlockSpec((H, H), lambda i: (0, 0)),
          pl.BlockSpec((H, H), lambda i: (0, 0)),
          pl.BlockSpec((H, H), lambda i: (0, 0)),
          pl.BlockSpec((H, H), lambda i: (0, 0)),
          pl.BlockSpec((3, H), lambda i: (0, 0)),
          pl.BlockSpec((3, H), lambda i: (0, 0)),
      ],
      out_specs=pl.BlockSpec((nb, H), lambda i: (i, 0)),
      out_shape=jax.ShapeDtypeStruct((N, H), _f32),
  )(hA, hA, hB, hB, wir, wiz, win, whr, whz, whn, bi, bh)


def kernel(x, edge_index, edge_weight, pre_W, pre_b, g1_lin, g1_root, g1_b,
           g2_lin, g2_root, g2_b, W_ih, W_hh, b_ih, b_hh):
  xs = x.reshape(T, N, F)
  src = edge_index[0]
  dst = edge_index[1]

  pad = EPAD - E
  src4 = jnp.pad(src, (0, pad)).reshape(32, EPAD // (32 * L), L)
  dst4 = jnp.pad(dst, (0, pad)).reshape(32, EPAD // (32 * L), L)
  ew4 = jnp.pad(edge_weight, (0, pad)).reshape(32, EPAD // (32 * L), L)

  degp = _deg_call(dst4, ew4)
  dinv = _dinv_call(degp)
  w4 = _w_call(src4, dst4, ew4, dinv.reshape(NPAD))
  w = w4.reshape(EPAD)[:E]

  src3 = src.reshape(NS, _NCH, _CH)
  dst3 = dst.reshape(NS, _NCH, _CH)
  w3 = w.reshape(NS, _NCH, _CH)

  pb = pre_b.reshape(1, H)
  h0A = _pre_call(xs, pre_W, pb, 0).reshape(NC * N, HQ)
  h0B = _pre_call(xs, pre_W, pb, 1).reshape(NC * N, HQ)

  lin1 = jnp.kron(jnp.eye(4, dtype=_f32), g1_lin)
  root1 = jnp.kron(jnp.eye(4, dtype=_f32), g1_root)
  b1 = jnp.tile(g1_b, 4).reshape(1, 128)
  lin2 = jnp.kron(jnp.eye(4, dtype=_f32), g2_lin)
  root2 = jnp.kron(jnp.eye(4, dtype=_f32), g2_root)
  b2 = jnp.tile(g2_b, 4).reshape(1, 128)

  nrows = NC * N * HQ // 128

  def layer(hA, hB, lin, root, bias):
    aggA = _scat_call(hA, src3, dst3, w3)
    aggB = _scat_call(hB, src3, dst3, w3)
    oA = _conv_call(aggA.reshape(nrows, 128), hA.reshape(nrows, 128),
                    lin, root, bias).reshape(NC * N, HQ)
    oB = _conv_call(aggB.reshape(nrows, 128), hB.reshape(nrows, 128),
                    lin, root, bias).reshape(NC * N, HQ)
    return oA, oB

  h1A, h1B = layer(h0A, h0B, lin1, root1, b1)
  h2A, h2B = layer(h1A, h1B, lin2, root2, b2)

  feats = _gru_call(h2A, h2B, W_ih, W_hh, b_ih, b_hh)
  return feats.reshape(1, N, H)


# speedup vs baseline: 1.2581x; 1.2581x over previous
import functools

import jax
import jax.numpy as jnp
from jax import lax
from jax.experimental import pallas as pl
from jax.experimental.pallas import tpu as pltpu
from jax.experimental.pallas import tpu_sc as plsc

N = 10000
NPAD = 10240
E = 160000
EPAD = 160256
T = 12
H = 32
F = 128
HQ = 96
TQ = 3
NC, NS, L = 2, 16, 16

@functools.cache
def _mesh():
  return plsc.VectorSubcoreMesh(
      core_axis_name="c", subcore_axis_name="s", num_cores=NC, num_subcores=NS)


_f32 = jnp.float32
_i32 = jnp.int32

_SC_PARAMS = pltpu.CompilerParams(
    needs_layout_passes=False, use_tc_tiling_on_sc=False)


def _deg_body(dst4, ew4, degp, dstv, ewv, degv, kbuf):
  cid = lax.axis_index("c")
  sid = lax.axis_index("s")
  wid = cid * NS + sid
  pltpu.sync_copy(dst4.at[wid], dstv)
  pltpu.sync_copy(ew4.at[wid], ewv)

  z16 = jnp.zeros((L,), _f32)

  @pl.loop(0, NPAD // L)
  def _zero(i):
    degv[pl.ds(i * L, L)] = z16

  lane = lax.iota(_i32, L)
  nxt = jnp.minimum(lane + 1, L - 1)

  @pl.loop(0, EPAD // (32 * L))
  def _chunk(i):
    keys = dstv[i]
    vals = ewv[i]
    sk, sv = plsc.sort_key_val(keys, vals)
    cs = plsc.cumsum(sv)
    kbuf[...] = sk
    knext = plsc.load_gather(kbuf, [nxt])
    is_last = (sk != knext) | (lane == L - 1)
    plsc.addupdate_scatter(degv, [sk], cs, mask=is_last)
    notfin = is_last & (lane < L - 1)
    plsc.addupdate_scatter(degv, [knext], -cs, mask=notfin)

  pltpu.sync_copy(degv, degp.at[wid])


def _deg_call(dst4, ew4):
  return pl.kernel(
      _deg_body,
      out_type=jax.ShapeDtypeStruct((32, NPAD), _f32),
      mesh=_mesh(),
      compiler_params=_SC_PARAMS,
      scratch_types=[
          pltpu.VMEM((EPAD // (32 * L), L), _i32),
          pltpu.VMEM((EPAD // (32 * L), L), _f32),
          pltpu.VMEM((NPAD,), _f32),
          pltpu.VMEM((L,), _i32),
      ],
  )(dst4, ew4)


def _w_body(src4, dst4, ew4, dinv_hbm, w4, srcv, dstv, ewv, dinvv, wv):
  cid = lax.axis_index("c")
  sid = lax.axis_index("s")
  wid = cid * NS + sid
  pltpu.sync_copy(dinv_hbm, dinvv)
  pltpu.sync_copy(src4.at[wid], srcv)
  pltpu.sync_copy(dst4.at[wid], dstv)
  pltpu.sync_copy(ew4.at[wid], ewv)

  @pl.loop(0, EPAD // (32 * L))
  def _chunk(i):
    s = srcv[i]
    d = dstv[i]
    e = ewv[i]
    ws = plsc.load_gather(dinvv, [s])
    wd = plsc.load_gather(dinvv, [d])
    wv[i] = ws * e * wd

  pltpu.sync_copy(wv, w4.at[wid])


def _w_call(src4, dst4, ew4, dinv):
  return pl.kernel(
      _w_body,
      out_type=jax.ShapeDtypeStruct((32, EPAD // (32 * L), L), _f32),
      mesh=_mesh(),
      compiler_params=_SC_PARAMS,
      scratch_types=[
          pltpu.VMEM((EPAD // (32 * L), L), _i32),
          pltpu.VMEM((EPAD // (32 * L), L), _i32),
          pltpu.VMEM((EPAD // (32 * L), L), _f32),
          pltpu.VMEM((NPAD,), _f32),
          pltpu.VMEM((EPAD // (32 * L), L), _f32),
      ],
  )(src4, dst4, ew4, dinv)


_EPT = E // NS
_CH = 80
_NCH = _EPT // _CH


def _scat_body(h2_hbm, src3, dst3, w3, out_hbm,
               srcv, dstv, wv, raw0, raw1, scl0, scl1,
               ig0, ig1, is0, is1, zbuf, agg_sh,
               gsem0, gsem1, ssem0, ssem1):
  cid = lax.axis_index("c")
  sid = lax.axis_index("s")
  raws = (raw0, raw1)
  scls = (scl0, scl1)
  igs = (ig0, ig1)
  iss = (is0, is1)
  gsems = (gsem0, gsem1)
  ssems = (ssem0, ssem1)
  hoff = cid * N

  z16 = jnp.zeros((L,), _f32)

  @pl.loop(0, 25)
  def _zero(i):
    for j in range(HQ // L):
      zbuf[i, pl.ds(j * L, L)] = z16

  @pl.loop(0, 25)
  def _zcopy(r):
    pltpu.sync_copy(zbuf, agg_sh.at[pl.ds(sid * 625 + r * 25, 25)])

  pltpu.sync_copy(src3.at[sid], srcv)
  pltpu.sync_copy(dst3.at[sid], dstv)
  pltpu.sync_copy(w3.at[sid], wv)
  plsc.subcore_barrier()

  def start_gather(ci, b):
    for p in range(_CH // L):
      igs[b][pl.ds(p * L, L)] = srcv[ci, pl.ds(p * L, L)] + hoff
    pltpu.make_async_copy(h2_hbm.at[igs[b]], raws[b], gsems[b]).start()

  def wait_gather(b):
    pltpu.make_async_copy(h2_hbm.at[igs[b]], raws[b], gsems[b]).wait()

  def scale_fast(ci, b):
    for e in range(_CH):
      wsp = plsc.load_gather(
          wv, [jnp.full((L,), ci, _i32), jnp.full((L,), e, _i32)])
      for j in range(HQ // L):
        sl = pl.ds(j * L, L)
        scls[b][e, sl] = raws[b][e, sl] * wsp

  def scale(ci, b):
    @pl.loop(0, _CH, unroll=8)
    def _edges(e):
      wsp = plsc.load_gather(
          wv, [jnp.full((L,), ci, _i32), jnp.full((L,), e, _i32)])
      for j in range(HQ // L):
        sl = pl.ds(j * L, L)
        scls[b][e, sl] = raws[b][e, sl] * wsp

  def start_scatter(ci, b):
    for p in range(_CH // L):
      iss[b][pl.ds(p * L, L)] = dstv[ci, pl.ds(p * L, L)]
    pltpu.async_copy(scls[b], agg_sh.at[iss[b]], ssems[b], add=True)

  def wait_scatter(b):
    pltpu.make_async_copy(scls[b], agg_sh.at[iss[b]], ssems[b]).wait()

  start_gather(0, 0)
  start_gather(1, 1)
  for b in (0, 1):
    wait_gather(b)
    scale(b, b)
    start_gather(b + 2, b)
    start_scatter(b, b)

  @pl.loop(1, (_NCH - 5) // 2 + 1)
  def _main(g):
    for b in (0, 1):
      ci = 2 * g + b
      wait_gather(b)
      wait_scatter(b)
      scale_fast(ci, b)
      start_gather(ci + 2, b)
      start_scatter(ci, b)

  wait_gather(0)
  wait_scatter(0)
  scale(_NCH - 3, 0)
  start_gather(_NCH - 1, 0)
  start_scatter(_NCH - 3, 0)

  wait_gather(1)
  wait_scatter(1)
  scale(_NCH - 2, 1)
  start_scatter(_NCH - 2, 1)

  wait_gather(0)
  wait_scatter(0)
  scale(_NCH - 1, 0)
  start_scatter(_NCH - 1, 0)

  wait_scatter(0)
  wait_scatter(1)
  plsc.subcore_barrier()

  pltpu.sync_copy(agg_sh.at[pl.ds(sid * 625, 625)],
                  out_hbm.at[pl.ds(cid * N + sid * 625, 625)])


def _scat_call(h2, src3, dst3, w3):
  return pl.kernel(
      _scat_body,
      out_type=jax.ShapeDtypeStruct((NC * N, HQ), _f32),
      mesh=_mesh(),
      compiler_params=_SC_PARAMS,
      scratch_types=[
          pltpu.VMEM((_NCH, _CH), _i32),
          pltpu.VMEM((_NCH, _CH), _i32),
          pltpu.VMEM((_NCH, _CH), _f32),
          pltpu.VMEM((_CH, HQ), _f32),
          pltpu.VMEM((_CH, HQ), _f32),
          pltpu.VMEM((_CH, HQ), _f32),
          pltpu.VMEM((_CH, HQ), _f32),
          pltpu.VMEM((_CH,), _i32),
          pltpu.VMEM((_CH,), _i32),
          pltpu.VMEM((_CH,), _i32),
          pltpu.VMEM((_CH,), _i32),
          pltpu.VMEM((25, HQ), _f32),
          pltpu.VMEM_SHARED((N, HQ), _f32),
          pltpu.SemaphoreType.DMA,
          pltpu.SemaphoreType.DMA,
          pltpu.SemaphoreType.DMA,
          pltpu.SemaphoreType.DMA,
      ],
  )(h2, src3, dst3, w3)


def _pre_body(x_ref, w_ref, b_ref, out_ref):
  w = w_ref[...]
  b = b_ref[...]
  for j in range(TQ):
    y = jnp.dot(x_ref[j], w, preferred_element_type=_f32) + b
    out_ref[:, j, :] = jnp.maximum(y, 0.0)


def _pre_call(xs, pre_W, pre_b, half):
  nb = 1000
  grid = (NC, N // nb)
  return pl.pallas_call(
      _pre_body,
      grid=grid,
      in_specs=[
          pl.BlockSpec((TQ, nb, F), lambda c, i: (2 * half + c, i, 0)),
          pl.BlockSpec((F, H), lambda c, i: (0, 0)),
          pl.BlockSpec((1, H), lambda c, i: (0, 0)),
      ],
      out_specs=pl.BlockSpec((nb, TQ, H),
                             lambda c, i: (c * (N // 1000) + i, 0, 0)),
      out_shape=jax.ShapeDtypeStruct((NC * N, TQ, H), _f32),
  )(xs, pre_W, pre_b)


def _dinv_body(degp_ref, out_ref):
  deg = jnp.sum(degp_ref[...], axis=0)
  good = deg > 0
  out_ref[...] = jnp.where(good, lax.rsqrt(jnp.maximum(deg, 1e-12)), 0.0)


def _dinv_call(degp):
  return pl.pallas_call(
      _dinv_body,
      out_shape=jax.ShapeDtypeStruct((NPAD // 128, 128), _f32),
  )(degp.reshape(32, NPAD // 128, 128))


def _conv_body(agg_ref, h_ref, wl_ref, wr_ref, b_ref, out_ref):
  y = (jnp.dot(agg_ref[...], wl_ref[...], preferred_element_type=_f32)
       + jnp.dot(h_ref[...], wr_ref[...], preferred_element_type=_f32)
       + b_ref[...])
  out_ref[...] = jnp.maximum(y, 0.0)


def _conv_call(agg128, h128, lin4, root4, b4):
  rb = 3000
  nrows = NC * N * HQ // 128
  return pl.pallas_call(
      _conv_body,
      grid=(nrows // rb,),
      in_specs=[
          pl.BlockSpec((rb, 128), lambda i: (i, 0)),
          pl.BlockSpec((rb, 128), lambda i: (i, 0)),
          pl.BlockSpec((128, 128), lambda i: (0, 0)),
          pl.BlockSpec((128, 128), lambda i: (0, 0)),
          pl.BlockSpec((1, 128), lambda i: (0, 0)),
      ],
      out_specs=pl.BlockSpec((rb, 128), lambda i: (i, 0)),
      out_shape=jax.ShapeDtypeStruct((nrows, 128), _f32),
  )(agg128, h128, lin4, root4, b4)


def _gru_body(q0_ref, q1_ref, q2_ref, q3_ref, wir_ref, wiz_ref, win_ref,
              whr_ref, whz_ref, whn_ref, bi_ref, bh_ref, out_ref):
  nb = q0_ref.shape[0]
  qs = [r[...].reshape(nb, TQ, H) for r in (q0_ref, q1_ref, q2_ref, q3_ref)]
  bi = bi_ref[...]
  bh = bh_ref[...]
  h = jnp.zeros((nb, H), _f32)
  for t in range(T):
    xt = qs[t // TQ][:, t % TQ, :]
    i_r = jnp.dot(xt, wir_ref[...], preferred_element_type=_f32) + bi[0:1]
    i_z = jnp.dot(xt, wiz_ref[...], preferred_element_type=_f32) + bi[1:2]
    i_n = jnp.dot(xt, win_ref[...], preferred_element_type=_f32) + bi[2:3]
    h_r = jnp.dot(h, whr_ref[...], preferred_element_type=_f32) + bh[0:1]
    h_z = jnp.dot(h, whz_ref[...], preferred_element_type=_f32) + bh[1:2]
    h_n = jnp.dot(h, whn_ref[...], preferred_element_type=_f32) + bh[2:3]
    r = jax.nn.sigmoid(i_r + h_r)
    z = jax.nn.sigmoid(i_z + h_z)
    c = jnp.tanh(i_n + r * h_n)
    h = (1.0 - z) * c + z * h
  out_ref[...] = h


def _gru_call(hA, hB, W_ih, W_hh, b_ih, b_hh):
  nb = 2000
  nblk = N // nb
  wir, wiz, win = W_ih[:, :H], W_ih[:, H:2 * H], W_ih[:, 2 * H:]
  whr, whz, whn = W_hh[:, :H], W_hh[:, H:2 * H], W_hh[:, 2 * H:]
  bi = b_ih.reshape(3, H)
  bh = b_hh.reshape(3, H)
  return pl.pallas_call(
      _gru_body,
      grid=(nblk,),
      in_specs=[
          pl.BlockSpec((nb, HQ), lambda i: (i, 0)),
          pl.BlockSpec((nb, HQ), lambda i: (i + nblk, 0)),
          pl.BlockSpec((nb, HQ), lambda i: (i, 0)),
          pl.BlockSpec((nb, HQ), lambda i: (i + nblk, 0)),
          pl.BlockSpec((H, H), lambda i: (0, 0)),
          pl.BlockSpec((H, H), lambda i: (0, 0)),
          pl.BlockSpec((H, H), lambda i: (0, 0)),
          pl.BlockSpec((H, H), lambda i: (0, 0)),
          pl.BlockSpec((H, H), lambda i: (0, 0)),
          pl.BlockSpec((H, H), lambda i: (0, 0)),
          pl.BlockSpec((3, H), lambda i: (0, 0)),
          pl.BlockSpec((3, H), lambda i: (0, 0)),
      ],
      out_specs=pl.BlockSpec((nb, H), lambda i: (i, 0)),
      out_shape=jax.ShapeDtypeStruct((N, H), _f32),
  )(hA, hA, hB, hB, wir, wiz, win, whr, whz, whn, bi, bh)


def kernel(x, edge_index, edge_weight, pre_W, pre_b, g1_lin, g1_root, g1_b,
           g2_lin, g2_root, g2_b, W_ih, W_hh, b_ih, b_hh):
  xs = x.reshape(T, N, F)
  src = edge_index[0]
  dst = edge_index[1]

  pad = EPAD - E
  src4 = jnp.pad(src, (0, pad)).reshape(32, EPAD // (32 * L), L)
  dst4 = jnp.pad(dst, (0, pad)).reshape(32, EPAD // (32 * L), L)
  ew4 = jnp.pad(edge_weight, (0, pad)).reshape(32, EPAD // (32 * L), L)

  degp = _deg_call(dst4, ew4)
  dinv = _dinv_call(degp)
  w4 = _w_call(src4, dst4, ew4, dinv.reshape(NPAD))
  w = w4.reshape(EPAD)[:E]

  src3 = src.reshape(NS, _NCH, _CH)
  dst3 = dst.reshape(NS, _NCH, _CH)
  w3 = w.reshape(NS, _NCH, _CH)

  pb = pre_b.reshape(1, H)
  h0A = _pre_call(xs, pre_W, pb, 0).reshape(NC * N, HQ)
  h0B = _pre_call(xs, pre_W, pb, 1).reshape(NC * N, HQ)

  lin1 = jnp.kron(jnp.eye(4, dtype=_f32), g1_lin)
  root1 = jnp.kron(jnp.eye(4, dtype=_f32), g1_root)
  b1 = jnp.tile(g1_b, 4).reshape(1, 128)
  lin2 = jnp.kron(jnp.eye(4, dtype=_f32), g2_lin)
  root2 = jnp.kron(jnp.eye(4, dtype=_f32), g2_root)
  b2 = jnp.tile(g2_b, 4).reshape(1, 128)

  nrows = NC * N * HQ // 128

  def layer(hA, hB, lin, root, bias):
    aggA = _scat_call(hA, src3, dst3, w3)
    aggB = _scat_call(hB, src3, dst3, w3)
    oA = _conv_call(aggA.reshape(nrows, 128), hA.reshape(nrows, 128),
                    lin, root, bias).reshape(NC * N, HQ)
    oB = _conv_call(aggB.reshape(nrows, 128), hB.reshape(nrows, 128),
                    lin, root, bias).reshape(NC * N, HQ)
    return oA, oB

  h1A, h1B = layer(h0A, h0B, lin1, root1, b1)
  h2A, h2B = layer(h1A, h1B, lin2, root2, b2)

  feats = _gru_call(h2A, h2B, W_ih, W_hh, b_ih, b_hh)
  return feats.reshape(1, N, H)


# speedup vs baseline: 1.2585x; 1.0003x over previous
import functools

import jax
import jax.numpy as jnp
from jax import lax
from jax.experimental import pallas as pl
from jax.experimental.pallas import tpu as pltpu
from jax.experimental.pallas import tpu_sc as plsc

N = 10000
NPAD = 10240
E = 160000
T = 12
H = 32
F = 128
HQ = 96
TQ = 3
NC, NS, L = 2, 16, 16
_EPW = E // 32
_FULL = _EPW // L

@functools.cache
def _mesh():
  return plsc.VectorSubcoreMesh(
      core_axis_name="c", subcore_axis_name="s", num_cores=NC, num_subcores=NS)


_f32 = jnp.float32
_i32 = jnp.int32

_SC_PARAMS = pltpu.CompilerParams(
    needs_layout_passes=False, use_tc_tiling_on_sc=False)


def _deg_body(dst_hbm, ew_hbm, degp, dstv, ewv, degv, kbuf):
  cid = lax.axis_index("c")
  sid = lax.axis_index("s")
  wid = cid * NS + sid
  base = wid * _EPW
  pltpu.sync_copy(dst_hbm.at[pl.ds(base, _EPW)], dstv.at[pl.ds(0, _EPW)])
  pltpu.sync_copy(ew_hbm.at[pl.ds(base, _EPW)], ewv.at[pl.ds(0, _EPW)])

  z16 = jnp.zeros((L,), _f32)

  @pl.loop(0, NPAD // L)
  def _zero(i):
    degv[pl.ds(i * L, L)] = z16

  lane = lax.iota(_i32, L)
  nxt = jnp.minimum(lane + 1, L - 1)
  tail_ok = lane < (_EPW - _FULL * L)

  def _accum(keys, vals):
    sk, sv = plsc.sort_key_val(keys, vals)
    cs = plsc.cumsum(sv)
    kbuf[...] = sk
    knext = plsc.load_gather(kbuf, [nxt])
    is_last = (sk != knext) | (lane == L - 1)
    plsc.addupdate_scatter(degv, [sk], cs, mask=is_last)
    notfin = is_last & (lane < L - 1)
    plsc.addupdate_scatter(degv, [knext], -cs, mask=notfin)

  @pl.loop(0, _FULL)
  def _chunk(i):
    _accum(dstv[pl.ds(i * L, L)], ewv[pl.ds(i * L, L)])

  _accum(jnp.where(tail_ok, dstv[pl.ds(_FULL * L, L)], NPAD - 1),
         jnp.where(tail_ok, ewv[pl.ds(_FULL * L, L)], 0.0))

  pltpu.sync_copy(degv, degp.at[wid])


def _deg_call(dst, ew):
  return pl.kernel(
      _deg_body,
      out_type=jax.ShapeDtypeStruct((32, NPAD), _f32),
      mesh=_mesh(),
      compiler_params=_SC_PARAMS,
      scratch_types=[
          pltpu.VMEM((_FULL * L + L,), _i32),
          pltpu.VMEM((_FULL * L + L,), _f32),
          pltpu.VMEM((NPAD,), _f32),
          pltpu.VMEM((L,), _i32),
      ],
  )(dst, ew)


def _w_body(src_hbm, dst_hbm, ew_hbm, dinv_hbm, w_hbm,
            srcv, dstv, ewv, dinvv, wv):
  cid = lax.axis_index("c")
  sid = lax.axis_index("s")
  wid = cid * NS + sid
  base = wid * _EPW
  pltpu.sync_copy(dinv_hbm, dinvv)
  pltpu.sync_copy(src_hbm.at[pl.ds(base, _EPW)], srcv.at[pl.ds(0, _EPW)])
  pltpu.sync_copy(dst_hbm.at[pl.ds(base, _EPW)], dstv.at[pl.ds(0, _EPW)])
  pltpu.sync_copy(ew_hbm.at[pl.ds(base, _EPW)], ewv.at[pl.ds(0, _EPW)])

  lane = lax.iota(_i32, L)
  tail_ok = lane < (_EPW - _FULL * L)

  @pl.loop(0, _FULL)
  def _chunk(i):
    sl = pl.ds(i * L, L)
    ws = plsc.load_gather(dinvv, [srcv[sl]])
    wd = plsc.load_gather(dinvv, [dstv[sl]])
    wv[sl] = ws * ewv[sl] * wd

  tl = pl.ds(_FULL * L, L)
  s = jnp.where(tail_ok, srcv[tl], 0)
  d = jnp.where(tail_ok, dstv[tl], 0)
  wv[tl] = plsc.load_gather(dinvv, [s]) * ewv[tl] * plsc.load_gather(dinvv, [d])

  pltpu.sync_copy(wv.at[pl.ds(0, _EPW)], w_hbm.at[pl.ds(base, _EPW)])


def _w_call(src, dst, ew, dinv):
  return pl.kernel(
      _w_body,
      out_type=jax.ShapeDtypeStruct((E,), _f32),
      mesh=_mesh(),
      compiler_params=_SC_PARAMS,
      scratch_types=[
          pltpu.VMEM((_FULL * L + L,), _i32),
          pltpu.VMEM((_FULL * L + L,), _i32),
          pltpu.VMEM((_FULL * L + L,), _f32),
          pltpu.VMEM((NPAD,), _f32),
          pltpu.VMEM((_FULL * L + L,), _f32),
      ],
  )(src, dst, ew, dinv)


_EPT = E // NS
_CH = 80
_NCH = _EPT // _CH


def _scat_body(h2_hbm, src3, dst3, w3, out_hbm,
               srcv, dstv, wv, raw0, raw1, scl0, scl1,
               ig0, ig1, is0, is1, zbuf, agg_sh,
               gsem0, gsem1, ssem0, ssem1):
  cid = lax.axis_index("c")
  sid = lax.axis_index("s")
  raws = (raw0, raw1)
  scls = (scl0, scl1)
  igs = (ig0, ig1)
  iss = (is0, is1)
  gsems = (gsem0, gsem1)
  ssems = (ssem0, ssem1)
  hoff = cid * N

  z16 = jnp.zeros((L,), _f32)

  @pl.loop(0, 25)
  def _zero(i):
    for j in range(HQ // L):
      zbuf[i, pl.ds(j * L, L)] = z16

  @pl.loop(0, 25)
  def _zcopy(r):
    pltpu.sync_copy(zbuf, agg_sh.at[pl.ds(sid * 625 + r * 25, 25)])

  pltpu.sync_copy(src3.at[sid], srcv)
  pltpu.sync_copy(dst3.at[sid], dstv)
  pltpu.sync_copy(w3.at[sid], wv)
  plsc.subcore_barrier()

  def start_gather(ci, b):
    for p in range(_CH // L):
      igs[b][pl.ds(p * L, L)] = srcv[ci, pl.ds(p * L, L)] + hoff
    pltpu.make_async_copy(h2_hbm.at[igs[b]], raws[b], gsems[b]).start()

  def wait_gather(b):
    pltpu.make_async_copy(h2_hbm.at[igs[b]], raws[b], gsems[b]).wait()

  def scale_fast(ci, b):
    for e in range(_CH):
      wsp = plsc.load_gather(
          wv, [jnp.full((L,), ci, _i32), jnp.full((L,), e, _i32)])
      for j in range(HQ // L):
        sl = pl.ds(j * L, L)
        scls[b][e, sl] = raws[b][e, sl] * wsp

  def scale(ci, b):
    @pl.loop(0, _CH, unroll=8)
    def _edges(e):
      wsp = plsc.load_gather(
          wv, [jnp.full((L,), ci, _i32), jnp.full((L,), e, _i32)])
      for j in range(HQ // L):
        sl = pl.ds(j * L, L)
        scls[b][e, sl] = raws[b][e, sl] * wsp

  def start_scatter(ci, b):
    for p in range(_CH // L):
      iss[b][pl.ds(p * L, L)] = dstv[ci, pl.ds(p * L, L)]
    pltpu.async_copy(scls[b], agg_sh.at[iss[b]], ssems[b], add=True)

  def wait_scatter(b):
    pltpu.make_async_copy(scls[b], agg_sh.at[iss[b]], ssems[b]).wait()

  start_gather(0, 0)
  start_gather(1, 1)
  for b in (0, 1):
    wait_gather(b)
    scale(b, b)
    start_gather(b + 2, b)
    start_scatter(b, b)

  @pl.loop(1, (_NCH - 5) // 2 + 1)
  def _main(g):
    for b in (0, 1):
      ci = 2 * g + b
      wait_gather(b)
      wait_scatter(b)
      scale_fast(ci, b)
      start_gather(ci + 2, b)
      start_scatter(ci, b)

  wait_gather(0)
  wait_scatter(0)
  scale(_NCH - 3, 0)
  start_gather(_NCH - 1, 0)
  start_scatter(_NCH - 3, 0)

  wait_gather(1)
  wait_scatter(1)
  scale(_NCH - 2, 1)
  start_scatter(_NCH - 2, 1)

  wait_gather(0)
  wait_scatter(0)
  scale(_NCH - 1, 0)
  start_scatter(_NCH - 1, 0)

  wait_scatter(0)
  wait_scatter(1)
  plsc.subcore_barrier()

  pltpu.sync_copy(agg_sh.at[pl.ds(sid * 625, 625)],
                  out_hbm.at[pl.ds(cid * N + sid * 625, 625)])


def _scat_call(h2, src3, dst3, w3):
  return pl.kernel(
      _scat_body,
      out_type=jax.ShapeDtypeStruct((NC * N, HQ), _f32),
      mesh=_mesh(),
      compiler_params=_SC_PARAMS,
      scratch_types=[
          pltpu.VMEM((_NCH, _CH), _i32),
          pltpu.VMEM((_NCH, _CH), _i32),
          pltpu.VMEM((_NCH, _CH), _f32),
          pltpu.VMEM((_CH, HQ), _f32),
          pltpu.VMEM((_CH, HQ), _f32),
          pltpu.VMEM((_CH, HQ), _f32),
          pltpu.VMEM((_CH, HQ), _f32),
          pltpu.VMEM((_CH,), _i32),
          pltpu.VMEM((_CH,), _i32),
          pltpu.VMEM((_CH,), _i32),
          pltpu.VMEM((_CH,), _i32),
          pltpu.VMEM((25, HQ), _f32),
          pltpu.VMEM_SHARED((N, HQ), _f32),
          pltpu.SemaphoreType.DMA,
          pltpu.SemaphoreType.DMA,
          pltpu.SemaphoreType.DMA,
          pltpu.SemaphoreType.DMA,
      ],
  )(h2, src3, dst3, w3)


def _pre_body(x_ref, w_ref, b_ref, out_ref):
  w = w_ref[...]
  b = b_ref[...]
  for j in range(TQ):
    y = jnp.dot(x_ref[j], w, preferred_element_type=_f32) + b
    out_ref[:, j, :] = jnp.maximum(y, 0.0)


def _pre_call(xs, pre_W, pre_b, half):
  nb = 1000
  grid = (NC, N // nb)
  return pl.pallas_call(
      _pre_body,
      grid=grid,
      in_specs=[
          pl.BlockSpec((TQ, nb, F), lambda c, i: (2 * half + c, i, 0)),
          pl.BlockSpec((F, H), lambda c, i: (0, 0)),
          pl.BlockSpec((1, H), lambda c, i: (0, 0)),
      ],
      out_specs=pl.BlockSpec((nb, TQ, H),
                             lambda c, i: (c * (N // 1000) + i, 0, 0)),
      out_shape=jax.ShapeDtypeStruct((NC * N, TQ, H), _f32),
  )(xs, pre_W, pre_b)


def _dinv_body(degp_ref, out_ref):
  deg = jnp.sum(degp_ref[...], axis=0)
  good = deg > 0
  out_ref[...] = jnp.where(good, lax.rsqrt(jnp.maximum(deg, 1e-12)), 0.0)


def _dinv_call(degp):
  return pl.pallas_call(
      _dinv_body,
      out_shape=jax.ShapeDtypeStruct((NPAD // 128, 128), _f32),
  )(degp.reshape(32, NPAD // 128, 128))


def _conv_body(agg_ref, h_ref, wl_ref, wr_ref, b_ref, out_ref):
  y = (jnp.dot(agg_ref[...], wl_ref[...], preferred_element_type=_f32)
       + jnp.dot(h_ref[...], wr_ref[...], preferred_element_type=_f32)
       + b_ref[...])
  out_ref[...] = jnp.maximum(y, 0.0)


def _conv_call(agg128, h128, lin4, root4, b4):
  rb = 3000
  nrows = NC * N * HQ // 128
  return pl.pallas_call(
      _conv_body,
      grid=(nrows // rb,),
      in_specs=[
          pl.BlockSpec((rb, 128), lambda i: (i, 0)),
          pl.BlockSpec((rb, 128), lambda i: (i, 0)),
          pl.BlockSpec((128, 128), lambda i: (0, 0)),
          pl.BlockSpec((128, 128), lambda i: (0, 0)),
          pl.BlockSpec((1, 128), lambda i: (0, 0)),
      ],
      out_specs=pl.BlockSpec((rb, 128), lambda i: (i, 0)),
      out_shape=jax.ShapeDtypeStruct((nrows, 128), _f32),
  )(agg128, h128, lin4, root4, b4)


def _gru_body(q0_ref, q1_ref, q2_ref, q3_ref, wir_ref, wiz_ref, win_ref,
              whr_ref, whz_ref, whn_ref, bi_ref, bh_ref, out_ref):
  nb = q0_ref.shape[0]
  qs = [r[...].reshape(nb, TQ, H) for r in (q0_ref, q1_ref, q2_ref, q3_ref)]
  bi = bi_ref[...]
  bh = bh_ref[...]
  h = jnp.zeros((nb, H), _f32)
  for t in range(T):
    xt = qs[t // TQ][:, t % TQ, :]
    i_r = jnp.dot(xt, wir_ref[...], preferred_element_type=_f32) + bi[0:1]
    i_z = jnp.dot(xt, wiz_ref[...], preferred_element_type=_f32) + bi[1:2]
    i_n = jnp.dot(xt, win_ref[...], preferred_element_type=_f32) + bi[2:3]
    h_r = jnp.dot(h, whr_ref[...], preferred_element_type=_f32) + bh[0:1]
    h_z = jnp.dot(h, whz_ref[...], preferred_element_type=_f32) + bh[1:2]
    h_n = jnp.dot(h, whn_ref[...], preferred_element_type=_f32) + bh[2:3]
    r = jax.nn.sigmoid(i_r + h_r)
    z = jax.nn.sigmoid(i_z + h_z)
    c = jnp.tanh(i_n + r * h_n)
    h = (1.0 - z) * c + z * h
  out_ref[...] = h


def _gru_call(hA, hB, W_ih, W_hh, b_ih, b_hh):
  nb = 2000
  nblk = N // nb
  wir, wiz, win = W_ih[:, :H], W_ih[:, H:2 * H], W_ih[:, 2 * H:]
  whr, whz, whn = W_hh[:, :H], W_hh[:, H:2 * H], W_hh[:, 2 * H:]
  bi = b_ih.reshape(3, H)
  bh = b_hh.reshape(3, H)
  return pl.pallas_call(
      _gru_body,
      grid=(nblk,),
      in_specs=[
          pl.BlockSpec((nb, HQ), lambda i: (i, 0)),
          pl.BlockSpec((nb, HQ), lambda i: (i + nblk, 0)),
          pl.BlockSpec((nb, HQ), lambda i: (i, 0)),
          pl.BlockSpec((nb, HQ), lambda i: (i + nblk, 0)),
          pl.BlockSpec((H, H), lambda i: (0, 0)),
          pl.BlockSpec((H, H), lambda i: (0, 0)),
          pl.BlockSpec((H, H), lambda i: (0, 0)),
          pl.BlockSpec((H, H), lambda i: (0, 0)),
          pl.BlockSpec((H, H), lambda i: (0, 0)),
          pl.BlockSpec((H, H), lambda i: (0, 0)),
          pl.BlockSpec((3, H), lambda i: (0, 0)),
          pl.BlockSpec((3, H), lambda i: (0, 0)),
      ],
      out_specs=pl.BlockSpec((nb, H), lambda i: (i, 0)),
      out_shape=jax.ShapeDtypeStruct((N, H), _f32),
  )(hA, hA, hB, hB, wir, wiz, win, whr, whz, whn, bi, bh)


def kernel(x, edge_index, edge_weight, pre_W, pre_b, g1_lin, g1_root, g1_b,
           g2_lin, g2_root, g2_b, W_ih, W_hh, b_ih, b_hh):
  xs = x.reshape(T, N, F)
  src = edge_index[0]
  dst = edge_index[1]

  degp = _deg_call(dst, edge_weight)
  dinv = _dinv_call(degp)
  w = _w_call(src, dst, edge_weight, dinv.reshape(NPAD))

  src3 = src.reshape(NS, _NCH, _CH)
  dst3 = dst.reshape(NS, _NCH, _CH)
  w3 = w.reshape(NS, _NCH, _CH)

  pb = pre_b.reshape(1, H)
  h0A = _pre_call(xs, pre_W, pb, 0).reshape(NC * N, HQ)
  h0B = _pre_call(xs, pre_W, pb, 1).reshape(NC * N, HQ)

  lin1 = jnp.kron(jnp.eye(4, dtype=_f32), g1_lin)
  root1 = jnp.kron(jnp.eye(4, dtype=_f32), g1_root)
  b1 = jnp.tile(g1_b, 4).reshape(1, 128)
  lin2 = jnp.kron(jnp.eye(4, dtype=_f32), g2_lin)
  root2 = jnp.kron(jnp.eye(4, dtype=_f32), g2_root)
  b2 = jnp.tile(g2_b, 4).reshape(1, 128)

  nrows = NC * N * HQ // 128

  def layer(hA, hB, lin, root, bias):
    aggA = _scat_call(hA, src3, dst3, w3)
    aggB = _scat_call(hB, src3, dst3, w3)
    oA = _conv_call(aggA.reshape(nrows, 128), hA.reshape(nrows, 128),
                    lin, root, bias).reshape(NC * N, HQ)
    oB = _conv_call(aggB.reshape(nrows, 128), hB.reshape(nrows, 128),
                    lin, root, bias).reshape(NC * N, HQ)
    return oA, oB

  h1A, h1B = layer(h0A, h0B, lin1, root1, b1)
  h2A, h2B = layer(h1A, h1B, lin2, root2, b2)

  feats = _gru_call(h2A, h2B, W_ih, W_hh, b_ih, b_hh)
  return feats.reshape(1, N, H)
